# baseline (speedup 1.0000x reference)
import functools

import jax
import jax.numpy as jnp
from jax import lax
from jax.experimental import pallas as pl
from jax.experimental.pallas import tpu as pltpu
from jax.experimental.pallas import tpu_sc as plsc

NC = 2
NS = 16
CH = 128
DW = 16


def _matmul_kernel(x_ref, w_ref, b_ref, o_ref):
    o_ref[0] = (
        jnp.dot(x_ref[...], w_ref[0], preferred_element_type=jnp.float32)
        + b_ref[0]
    )


def _finalize_kernel(a_ref, d_ref, o_ref):
    ssum = jnp.concatenate([a_ref[0], a_ref[1]], axis=-1)
    deg = jnp.maximum(d_ref[0, :, 0:1], 1.0)
    o_ref[...] = jnp.maximum(ssum / deg, 0.0)


M = 2


def _make_sc_agg(n_pad, dh, nch):
    rpt = n_pad // (NS * CH)
    ngrp = nch // M

    mesh = plsc.VectorSubcoreMesh(core_axis_name="c", subcore_axis_name="s")

    @functools.partial(
        pl.kernel,
        mesh=mesh,
        compiler_params=pltpu.CompilerParams(use_tc_tiling_on_sc=False),
        out_type=[
            jax.ShapeDtypeStruct((NC, n_pad, dh), jnp.float32),
            jax.ShapeDtypeStruct((NC, n_pad, DW), jnp.float32),
        ],
        scratch_types=[
            pltpu.VMEM((nch, CH), jnp.int32),
            pltpu.VMEM((nch, CH), jnp.int32),
            pltpu.VMEM((2 * M, CH, dh), jnp.float32),
            pltpu.VMEM((CH, DW), jnp.float32),
            pltpu.VMEM_SHARED((n_pad, dh), jnp.float32),
            pltpu.VMEM_SHARED((n_pad, DW), jnp.float32),
            pltpu.SemaphoreType.DMA,
            pltpu.SemaphoreType.DMA,
            pltpu.SemaphoreType.DMA,
            pltpu.SemaphoreType.DMA,
            pltpu.SemaphoreType.DMA,
            pltpu.SemaphoreType.DMA,
        ],
    )
    def sc_agg(h_hbm, src_hbm, dst_hbm, zrow_hbm, zdeg_hbm, ones_hbm,
               agg_out, deg_out,
               src_v, dst_v, bufs_v, ones_v, acc_sh, deg_sh,
               gsem_a, gsem_b, ssem_a, ssem_b, dsem, usem):
        c = lax.axis_index("c")
        s = lax.axis_index("s")

        def zbody(i, carry):
            r0 = (s * rpt + i) * CH
            pltpu.async_copy(zrow_hbm, acc_sh.at[pl.ds(r0, CH)], usem)
            pltpu.async_copy(zdeg_hbm, deg_sh.at[pl.ds(r0, CH)], usem)
            pltpu.make_async_copy(zrow_hbm, acc_sh.at[pl.ds(r0, CH)],
                                  usem).wait()
            pltpu.make_async_copy(zdeg_hbm, deg_sh.at[pl.ds(r0, CH)],
                                  usem).wait()
            return carry

        lax.fori_loop(0, rpt, zbody, 0)
        pltpu.sync_copy(src_hbm.at[s], src_v)
        pltpu.sync_copy(dst_hbm.at[s], dst_v)
        pltpu.sync_copy(ones_hbm, ones_v)
        plsc.subcore_barrier()

        def fire_gathers(grp, base, gsem):
            for i in range(M):
                pltpu.async_copy(h_hbm.at[c].at[src_v.at[grp * M + i]],
                                 bufs_v.at[base + i], gsem)

        def drain_scat(grp, base, ssem):
            for i in range(M):
                pltpu.make_async_copy(bufs_v.at[base + i],
                                      acc_sh.at[dst_v.at[grp * M + i]],
                                      ssem).wait()

        def half(grp, base, gsem, ssem):
            for i in range(M):
                pltpu.make_async_copy(h_hbm.at[c].at[src_v.at[grp * M + i]],
                                      bufs_v.at[base + i], gsem).wait()
            for i in range(M):
                pltpu.async_copy(bufs_v.at[base + i],
                                 acc_sh.at[dst_v.at[grp * M + i]],
                                 ssem, add=True)
                pltpu.async_copy(ones_v, deg_sh.at[dst_v.at[grp * M + i]],
                                 dsem, add=True)
            for i in range(M):
                pltpu.make_async_copy(ones_v,
                                      deg_sh.at[dst_v.at[grp * M + i]],
                                      dsem).wait()

        def body(t, carry):
            ga = 2 * t
            gb = 2 * t + 1

            @pl.when(t > 0)
            def _():
                drain_scat(ga - 1, M, ssem_b)

            fire_gathers(gb, M, gsem_b)
            half(ga, 0, gsem_a, ssem_a)

            drain_scat(ga, 0, ssem_a)

            @pl.when(t < ngrp // 2 - 1)
            def _():
                fire_gathers(ga + 2, 0, gsem_a)

            half(gb, M, gsem_b, ssem_b)
            return carry

        fire_gathers(0, 0, gsem_a)
        lax.fori_loop(0, ngrp // 2, body, 0)
        drain_scat(ngrp - 1, M, ssem_b)
        plsc.subcore_barrier()

        def wbody(i, carry):
            r0 = (s * rpt + i) * CH
            pltpu.sync_copy(acc_sh.at[pl.ds(r0, CH)],
                            agg_out.at[c, pl.ds(r0, CH)])
            pltpu.sync_copy(deg_sh.at[pl.ds(r0, CH)],
                            deg_out.at[c, pl.ds(r0, CH)])
            return carry

        lax.fori_loop(0, rpt, wbody, 0)

    return sc_agg


def kernel(x, edge_index, W, b):
    n, d_in = x.shape
    d = W.shape[1]
    dh = d // NC
    e = edge_index.shape[1]

    bn = 1000
    h = pl.pallas_call(
        _matmul_kernel,
        grid=(NC, n // bn),
        in_specs=[
            pl.BlockSpec((bn, d_in), lambda c, i: (i, 0)),
            pl.BlockSpec((1, d_in, dh), lambda c, i: (c, 0, 0)),
            pl.BlockSpec((1, 1, dh), lambda c, i: (c, 0, 0)),
        ],
        out_specs=pl.BlockSpec((1, bn, dh), lambda c, i: (c, i, 0)),
        out_shape=jax.ShapeDtypeStruct((NC, n, dh), jnp.float32),
    )(x, W.reshape(d_in, NC, dh).swapaxes(0, 1), b.reshape(NC, 1, dh))

    nch = -(-e // (NS * CH))
    nch = -(-nch // (2 * M)) * (2 * M)
    e_pad = NS * nch * CH
    n_pad = -(-(n + 1) // (NS * CH)) * (NS * CH)

    src = edge_index[0]
    dst = edge_index[1]
    pad = e_pad - e
    src3 = jnp.concatenate([src, jnp.zeros((pad,), jnp.int32)]).reshape(
        NS, nch, CH)
    dst3 = jnp.concatenate([dst, jnp.full((pad,), n, jnp.int32)]).reshape(
        NS, nch, CH)

    zrow = jnp.zeros((CH, dh), jnp.float32)
    zdeg = jnp.zeros((CH, DW), jnp.float32)
    ones = jnp.ones((CH, DW), jnp.float32)

    agg_p, deg_p = _make_sc_agg(n_pad, dh, nch)(h, src3, dst3, zrow, zdeg,
                                                ones)

    out = pl.pallas_call(
        _finalize_kernel,
        grid=(n // bn,),
        in_specs=[
            pl.BlockSpec((NC, bn, dh), lambda i: (0, i, 0)),
            pl.BlockSpec((1, bn, DW), lambda i: (0, i, 0)),
        ],
        out_specs=pl.BlockSpec((bn, d), lambda i: (i, 0)),
        out_shape=jax.ShapeDtypeStruct((n, d), jnp.float32),
    )(agg_p, deg_p)
    return out


# speedup vs baseline: 1.2024x; 1.2024x over previous
import functools

import jax
import jax.numpy as jnp
from jax import lax
from jax.experimental import pallas as pl
from jax.experimental.pallas import tpu as pltpu
from jax.experimental.pallas import tpu_sc as plsc

NC = 2
NS = 16
CH = 128
DP = 160


def _matmul_kernel(x_ref, w_ref, b_ref, o_ref):
    h = (jnp.dot(x_ref[...], w_ref[...], preferred_element_type=jnp.float32)
         + b_ref[...])
    bn = h.shape[0]
    o_ref[...] = jnp.concatenate(
        [h, jnp.ones((bn, DP - h.shape[1]), jnp.float32)],
        axis=-1).astype(jnp.bfloat16)


def _finalize_kernel(a_ref, o_ref):
    a = a_ref[0].astype(jnp.float32) + a_ref[1].astype(jnp.float32)
    deg = jnp.maximum(a[:, 128:129], 1.0)
    o_ref[...] = jnp.maximum(a[:, :128] / deg, 0.0)


def _make_sc_agg(n_pad, nch):
    rpt = n_pad // (NS * CH)

    mesh = plsc.VectorSubcoreMesh(core_axis_name="c", subcore_axis_name="s")

    @functools.partial(
        pl.kernel,
        mesh=mesh,
        compiler_params=pltpu.CompilerParams(use_tc_tiling_on_sc=False),
        out_type=jax.ShapeDtypeStruct((NC, n_pad, DP), jnp.bfloat16),
        scratch_types=[
            pltpu.VMEM((nch, CH), jnp.int32),
            pltpu.VMEM((nch, CH), jnp.int32),
            pltpu.VMEM((CH, DP), jnp.bfloat16),
            pltpu.VMEM_SHARED((n_pad, DP), jnp.bfloat16),
            pltpu.SemaphoreType.DMA,
        ],
    )
    def sc_agg(h_hbm, src_hbm, dst_hbm, zrow_hbm,
               agg_out, src_v, dst_v, rows_v, acc_sh, sem):
        c = lax.axis_index("c")
        s = lax.axis_index("s")

        def zbody(i, carry):
            r0 = (s * rpt + i) * CH
            pltpu.sync_copy(zrow_hbm, acc_sh.at[pl.ds(r0, CH)])
            return carry

        lax.fori_loop(0, rpt, zbody, 0)

        pltpu.sync_copy(src_hbm.at[c, s], src_v)
        pltpu.sync_copy(dst_hbm.at[c, s], dst_v)
        plsc.subcore_barrier()

        def ebody(j, carry):
            pltpu.async_copy(h_hbm.at[src_v.at[j]], rows_v, sem).wait()
            pltpu.sync_copy(rows_v, acc_sh.at[dst_v.at[j]], add=True)
            return carry

        lax.fori_loop(0, nch, ebody, 0)
        plsc.subcore_barrier()

        def wbody(i, carry):
            r0 = (s * rpt + i) * CH
            pltpu.sync_copy(acc_sh.at[pl.ds(r0, CH)],
                            agg_out.at[c, pl.ds(r0, CH)])
            return carry

        lax.fori_loop(0, rpt, wbody, 0)

    return sc_agg


def kernel(x, edge_index, W, b):
    n, d_in = x.shape
    d = W.shape[1]
    e = edge_index.shape[1]

    bn = 1000
    h = pl.pallas_call(
        _matmul_kernel,
        grid=(n // bn,),
        in_specs=[
            pl.BlockSpec((bn, d_in), lambda i: (i, 0)),
            pl.BlockSpec((d_in, d), lambda i: (0, 0)),
            pl.BlockSpec((1, d), lambda i: (0, 0)),
        ],
        out_specs=pl.BlockSpec((bn, DP), lambda i: (i, 0)),
        out_shape=jax.ShapeDtypeStruct((n, DP), jnp.bfloat16),
    )(x, W, b.reshape(1, d))

    nw = NC * NS
    nch = -(-e // (nw * CH))
    e_pad = nw * nch * CH
    n_pad = -(-(n + 1) // (NS * CH)) * (NS * CH)

    src = edge_index[0]
    dst = edge_index[1]
    pad = e_pad - e
    src3 = jnp.concatenate([src, jnp.zeros((pad,), jnp.int32)]).reshape(
        NC, NS, nch, CH)
    dst3 = jnp.concatenate([dst, jnp.full((pad,), n, jnp.int32)]).reshape(
        NC, NS, nch, CH)

    zrow = jnp.zeros((CH, DP), jnp.bfloat16)

    agg_p = _make_sc_agg(n_pad, nch)(h, src3, dst3, zrow)

    out = pl.pallas_call(
        _finalize_kernel,
        grid=(n // bn,),
        in_specs=[
            pl.BlockSpec((NC, bn, DP), lambda i: (0, i, 0)),
        ],
        out_specs=pl.BlockSpec((bn, d), lambda i: (i, 0)),
        out_shape=jax.ShapeDtypeStruct((n, d), jnp.float32),
    )(agg_p)
    return out
